# Initial kernel scaffold; baseline (speedup 1.0000x reference)
import functools
import math

import jax
import jax.numpy as jnp
from jax import lax
from jax.experimental import pallas as pl
from jax.experimental.pallas import tpu as pltpu
from jax.experimental.pallas import tpu_sc as plsc

N = 10000
E = 160000
D = 128
MSG = 24
ED = 16
L = 6
TW = 32
CH = 128
NCHUNK = E // CH
NC, NS = 2, 16
NW = NC * NS
EB = 4000
NB = 2000

_f32 = jnp.float32

_mesh = plsc.VectorSubcoreMesh(
    core_axis_name="c", subcore_axis_name="s", num_cores=NC, num_subcores=NS)


@functools.partial(
    pl.kernel,
    out_type=(jax.ShapeDtypeStruct((NCHUNK, CH, TW), _f32),
              jax.ShapeDtypeStruct((NCHUNK, CH, TW), _f32)),
    mesh=_mesh,
    scratch_types=[pltpu.VMEM((CH,), jnp.int32),
                   pltpu.VMEM((CH,), jnp.int32),
                   pltpu.VMEM((CH, TW), _f32),
                   pltpu.VMEM((CH, TW), _f32),
                   pltpu.SemaphoreType.DMA,
                   pltpu.SemaphoreType.DMA],
)
def _gather_k(tcol_hbm, trow_hbm, col_hbm, row_hbm, gcol_hbm, grow_hbm,
              idxc_v, idxr_v, bufc_v, bufr_v, sem1, sem2):
    cid = lax.axis_index("c")
    sid = lax.axis_index("s")
    wid = sid * NC + cid
    nj = (NCHUNK + NW - 1) // NW

    def body(j, carry):
        c = wid + j * NW

        @pl.when(c < NCHUNK)
        def _():
            pltpu.sync_copy(col_hbm.at[c], idxc_v)
            pltpu.sync_copy(row_hbm.at[c], idxr_v)
            d1 = pltpu.async_copy(tcol_hbm.at[idxc_v], bufc_v, sem1)
            d2 = pltpu.async_copy(trow_hbm.at[idxr_v], bufr_v, sem2)
            d1.wait()
            d2.wait()
            pltpu.sync_copy(bufc_v, gcol_hbm.at[c])
            pltpu.sync_copy(bufr_v, grow_hbm.at[c])
        return carry

    lax.fori_loop(0, nj, body, 0)


@functools.partial(
    pl.kernel,
    out_type=jax.ShapeDtypeStruct((NC, N, TW), _f32),
    mesh=_mesh,
    scratch_types=[pltpu.VMEM((CH,), jnp.int32),
                   pltpu.VMEM((CH, TW), _f32),
                   pltpu.VMEM_SHARED((N, TW), _f32)],
)
def _scatter_k(o_hbm, col_hbm, zero_hbm, p_hbm, idx_v, buf_v, acc_sh):
    cid = lax.axis_index("c")
    sid = lax.axis_index("s")
    rows_per = N // NS
    r0 = sid * rows_per
    pltpu.sync_copy(zero_hbm.at[pl.ds(r0, rows_per)],
                    acc_sh.at[pl.ds(r0, rows_per)])
    plsc.subcore_barrier()
    half = NCHUNK // NC
    nj = (half + NS - 1) // NS

    def body(j, carry):
        jj = sid + j * NS

        @pl.when(jj < half)
        def _():
            c = cid * half + jj
            pltpu.sync_copy(col_hbm.at[c], idx_v)
            pltpu.sync_copy(o_hbm.at[c], buf_v)
            pltpu.sync_copy(buf_v, acc_sh.at[idx_v], add=True)
        return carry

    lax.fori_loop(0, nj, body, 0)
    plsc.subcore_barrier()
    pltpu.sync_copy(acc_sh.at[pl.ds(r0, rows_per)],
                    p_hbm.at[cid, pl.ds(r0, rows_per)])


def _silu(x):
    return x / (1.0 + jnp.exp(-x))


def _dot(a, b):
    return jnp.dot(a, b, preferred_element_type=_f32)


def _edge_body(first, coords, refs):
    if first:
        (gcol, grow, freqs, w1c, w1r, b1, w2, b2, cw1, cb1, cw2,
         o_ref, ea_out) = refs
    else:
        (gcol, grow, ea_in, w1c, w1r, b1, w2, b2, cw1, cb1, cw2,
         o_ref) = refs
    A = gcol[:, :MSG]
    posc = gcol[:, MSG:MSG + 3]
    B = grow[:, :MSG]
    posr = grow[:, MSG:MSG + 3]
    cd = posr - posc
    radial = jnp.sum(cd * cd, axis=1, keepdims=True)
    if first:
        dist = jnp.sqrt(radial)
        ang = dist * freqs[...]
        ea = jnp.concatenate([jnp.sin(ang), jnp.cos(ang)], axis=1)
        ea_out[...] = ea
    else:
        ea = ea_in[...]
    z1 = A + B + _dot(ea, w1c[...]) + radial * w1r[...] + b1[...]
    m = _silu(z1)
    m = _silu(_dot(m, w2[...]) + b2[...])
    if coords:
        cw = _dot(_silu(_dot(m, cw1[...]) + cb1[...]), cw2[...])
        wcd = cd * cw
    else:
        wcd = jnp.zeros((m.shape[0], 3), _f32)
    ones = jnp.ones((m.shape[0], 1), _f32)
    pad = jnp.zeros((m.shape[0], TW - MSG - 4), _f32)
    o_ref[...] = jnp.concatenate([m, wcd, ones, pad], axis=1)


def _make_edge_call(first, coords):
    def body(*refs):
        _edge_body(first, coords, refs)

    grid = E // EB
    eblk = pl.BlockSpec((EB, TW), lambda i: (i, 0))
    eablk = pl.BlockSpec((EB, ED), lambda i: (i, 0))

    def w(shape):
        return pl.BlockSpec(shape, lambda i: tuple(0 for _ in shape))

    in_specs = [eblk, eblk]
    if first:
        in_specs.append(w((1, ED // 2)))
    else:
        in_specs.append(eablk)
    in_specs += [w((ED, MSG)), w((1, MSG)), w((1, MSG)), w((MSG, MSG)),
                 w((1, MSG)), w((MSG, MSG)), w((1, MSG)), w((MSG, 1))]
    out_shape = [jax.ShapeDtypeStruct((E, TW), _f32)]
    out_specs = [eblk]
    if first:
        out_shape.append(jax.ShapeDtypeStruct((E, ED), _f32))
        out_specs.append(eablk)
    return pl.pallas_call(
        body, grid=(grid,), in_specs=in_specs,
        out_specs=out_specs if len(out_specs) > 1 else out_specs[0],
        out_shape=out_shape if len(out_shape) > 1 else out_shape[0])


_edge_first = _make_edge_call(True, True)
_edge_mid = _make_edge_call(False, True)
_edge_last = _make_edge_call(False, False)


def _node_body(tables, refs):
    if tables:
        (h_ref, pos_ref, p_ref, w1h, w1m, nb1, w2, nb2, g, b, wa, wb,
         h_out, pos_out, tcol_out, trow_out) = refs
    else:
        (h_ref, pos_ref, p_ref, w1h, w1m, nb1, w2, nb2, g, b,
         h_out, pos_out) = refs
    S = p_ref[0] + p_ref[1]
    msum = S[:, :MSG]
    coordagg = S[:, MSG:MSG + 3]
    cnt = S[:, MSG + 3:MSG + 4]
    agg = msum / jnp.maximum(cnt, 1.0)
    h = h_ref[...]
    z = _dot(h, w1h[...]) + _dot(agg, w1m[...]) + nb1[...]
    h2 = h + _dot(_silu(z), w2[...]) + nb2[...]
    mu = jnp.mean(h2, axis=-1, keepdims=True)
    hc = h2 - mu
    var = jnp.mean(hc * hc, axis=-1, keepdims=True)
    hn = hc / jnp.sqrt(var + 1e-5) * g[...] + b[...]
    h_out[...] = hn
    pos_new = pos_ref[...] + coordagg
    pos_out[...] = pos_new
    if tables:
        padz = jnp.zeros((hn.shape[0], TW - MSG - 3), _f32)
        tcol_out[...] = jnp.concatenate([_dot(hn, wa[...]), pos_new, padz],
                                        axis=1)
        trow_out[...] = jnp.concatenate([_dot(hn, wb[...]), pos_new, padz],
                                        axis=1)


def _make_node_call(tables):
    def body(*refs):
        _node_body(tables, refs)

    grid = N // NB
    hblk = pl.BlockSpec((NB, D), lambda i: (i, 0))
    pblk = pl.BlockSpec((NB, 3), lambda i: (i, 0))
    tblk = pl.BlockSpec((NB, TW), lambda i: (i, 0))
    partblk = pl.BlockSpec((NC, NB, TW), lambda i: (0, i, 0))

    def w(shape):
        return pl.BlockSpec(shape, lambda i: tuple(0 for _ in shape))

    in_specs = [hblk, pblk, partblk, w((D, MSG)), w((MSG, MSG)), w((1, MSG)),
                w((MSG, D)), w((1, D)), w((1, D)), w((1, D))]
    out_shape = [jax.ShapeDtypeStruct((N, D), _f32),
                 jax.ShapeDtypeStruct((N, 3), _f32)]
    out_specs = [hblk, pblk]
    if tables:
        in_specs += [w((D, MSG)), w((D, MSG))]
        out_shape += [jax.ShapeDtypeStruct((N, TW), _f32),
                      jax.ShapeDtypeStruct((N, TW), _f32)]
        out_specs += [tblk, tblk]
    return pl.pallas_call(
        body, grid=(grid,), in_specs=in_specs,
        out_specs=out_specs, out_shape=out_shape)


_node_mid = _make_node_call(True)
_node_last = _make_node_call(False)


def _prologue_body(h_ref, pos_ref, ew, eb, wa, wb, h_out, tcol_out, trow_out):
    h0 = _dot(h_ref[...], ew[...]) + eb[...]
    h_out[...] = h0
    padz = jnp.zeros((h0.shape[0], TW - MSG - 3), _f32)
    pos = pos_ref[...]
    tcol_out[...] = jnp.concatenate([_dot(h0, wa[...]), pos, padz], axis=1)
    trow_out[...] = jnp.concatenate([_dot(h0, wb[...]), pos, padz], axis=1)


def _make_prologue():
    grid = N // NB
    hblk = pl.BlockSpec((NB, D), lambda i: (i, 0))
    pblk = pl.BlockSpec((NB, 3), lambda i: (i, 0))
    tblk = pl.BlockSpec((NB, TW), lambda i: (i, 0))

    def w(shape):
        return pl.BlockSpec(shape, lambda i: tuple(0 for _ in shape))

    return pl.pallas_call(
        _prologue_body, grid=(grid,),
        in_specs=[hblk, pblk, w((D, D)), w((1, D)), w((D, MSG)), w((D, MSG))],
        out_specs=[hblk, tblk, tblk],
        out_shape=[jax.ShapeDtypeStruct((N, D), _f32),
                   jax.ShapeDtypeStruct((N, TW), _f32),
                   jax.ShapeDtypeStruct((N, TW), _f32)])


_prologue = _make_prologue()


def kernel(h, pos, edge_index, embed_W, embed_b, msg_W1, msg_b1, msg_W2,
           msg_b2, coord_W1, coord_b1, coord_W2, node_W1, node_b1, node_W2,
           node_b2, ln_g, ln_b):
    row = edge_index[0].reshape(NCHUNK, CH)
    col = edge_index[1].reshape(NCHUNK, CH)
    freqs = jnp.exp(-math.log(10000.0)
                    * jnp.arange(0, ED, 2, dtype=_f32) / ED).reshape(1, ED // 2)
    zeros_tbl = jnp.zeros((N, TW), _f32)

    W1a = msg_W1[:, :D]
    W1b = msg_W1[:, D:2 * D]
    W1c = jnp.concatenate([msg_W1[:, 2 * D:2 * D + ED:2],
                           msg_W1[:, 2 * D + 1:2 * D + ED:2]], axis=1)
    w1r = msg_W1[:, 2 * D + ED].reshape(L, 1, MSG)
    b1 = msg_b1.reshape(L, 1, MSG)
    b2 = msg_b2.reshape(L, 1, MSG)
    cb1 = coord_b1.reshape(L - 1, 1, MSG)
    nW1h = node_W1[:, :D]
    nW1m = node_W1[:, D:]
    nb1 = node_b1.reshape(L, 1, MSG)
    nb2 = node_b2.reshape(L, 1, D)
    g = ln_g.reshape(L, 1, D)
    bb = ln_b.reshape(L, 1, D)

    hcur, tcol, trow = _prologue(h, pos, embed_W, embed_b.reshape(1, D),
                                 W1a[0], W1b[0])
    poscur = pos
    ea = None
    for l in range(L):
        gcol, grow = _gather_k(tcol, trow, col, row)
        gcol = gcol.reshape(E, TW)
        grow = grow.reshape(E, TW)
        if l == 0:
            O, ea = _edge_first(gcol, grow, freqs, W1c[l], w1r[l], b1[l],
                                msg_W2[l], b2[l], coord_W1[l], cb1[l],
                                coord_W2[l])
        elif l < L - 1:
            O = _edge_mid(gcol, grow, ea, W1c[l], w1r[l], b1[l],
                          msg_W2[l], b2[l], coord_W1[l], cb1[l], coord_W2[l])
        else:
            O = _edge_last(gcol, grow, ea, W1c[l], w1r[l], b1[l],
                           msg_W2[l], b2[l], coord_W1[0], cb1[0], coord_W2[0])
        P = _scatter_k(O.reshape(NCHUNK, CH, TW), col, zeros_tbl)
        if l < L - 1:
            hcur, poscur, tcol, trow = _node_mid(
                hcur, poscur, P, nW1h[l], nW1m[l], nb1[l], node_W2[l],
                nb2[l], g[l], bb[l], W1a[l + 1], W1b[l + 1])
        else:
            hcur, poscur = _node_last(
                hcur, poscur, P, nW1h[l], nW1m[l], nb1[l], node_W2[l],
                nb2[l], g[l], bb[l])
    return hcur, poscur


# speedup vs baseline: 3.7214x; 3.7214x over previous
import functools
import math

import jax
import jax.numpy as jnp
from jax import lax
from jax.experimental import pallas as pl
from jax.experimental.pallas import tpu as pltpu
from jax.experimental.pallas import tpu_sc as plsc

N = 10000
E = 160000
D = 128
MSG = 24
ED = 16
L = 6
TW = 32
CH = 128
NCHUNK = E // CH
NC, NS = 2, 16
NW = NC * NS
EB = 4000
NB = 2000

_f32 = jnp.float32

_mesh = plsc.VectorSubcoreMesh(
    core_axis_name="c", subcore_axis_name="s", num_cores=NC, num_subcores=NS)


@functools.partial(
    pl.kernel,
    out_type=(jax.ShapeDtypeStruct((NCHUNK, CH, TW), _f32),
              jax.ShapeDtypeStruct((NCHUNK, CH, TW), _f32)),
    mesh=_mesh,
    scratch_types=[pltpu.VMEM((CH,), jnp.int32),
                   pltpu.VMEM((CH,), jnp.int32),
                   pltpu.VMEM((CH, TW), _f32),
                   pltpu.VMEM((CH, TW), _f32),
                   pltpu.SemaphoreType.DMA,
                   pltpu.SemaphoreType.DMA],
    compiler_params=pltpu.CompilerParams(use_tc_tiling_on_sc=False),
)
def _gather_k(tcol_hbm, trow_hbm, col_hbm, row_hbm, gcol_hbm, grow_hbm,
              idxc_v, idxr_v, bufc_v, bufr_v, sem1, sem2):
    cid = lax.axis_index("c")
    sid = lax.axis_index("s")
    wid = sid * NC + cid
    nj = (NCHUNK + NW - 1) // NW

    def body(j, carry):
        c = wid + j * NW

        @pl.when(c < NCHUNK)
        def _():
            pltpu.sync_copy(col_hbm.at[c], idxc_v)
            pltpu.sync_copy(row_hbm.at[c], idxr_v)
            d1 = pltpu.async_copy(tcol_hbm.at[idxc_v], bufc_v, sem1)
            d2 = pltpu.async_copy(trow_hbm.at[idxr_v], bufr_v, sem2)
            d1.wait()
            d2.wait()
            pltpu.sync_copy(bufc_v, gcol_hbm.at[c])
            pltpu.sync_copy(bufr_v, grow_hbm.at[c])
        return carry

    lax.fori_loop(0, nj, body, 0)


@functools.partial(
    pl.kernel,
    out_type=jax.ShapeDtypeStruct((NC, N, TW), _f32),
    mesh=_mesh,
    scratch_types=[pltpu.VMEM((CH,), jnp.int32),
                   pltpu.VMEM((CH, TW), _f32),
                   pltpu.VMEM_SHARED((N, TW), _f32)],
    compiler_params=pltpu.CompilerParams(use_tc_tiling_on_sc=False),
)
def _scatter_k(o_hbm, col_hbm, zero_hbm, p_hbm, idx_v, buf_v, acc_sh):
    cid = lax.axis_index("c")
    sid = lax.axis_index("s")
    rows_per = N // NS
    r0 = sid * rows_per
    pltpu.sync_copy(zero_hbm.at[pl.ds(r0, rows_per)],
                    acc_sh.at[pl.ds(r0, rows_per)])
    plsc.subcore_barrier()
    half = NCHUNK // NC
    nj = (half + NS - 1) // NS

    def body(j, carry):
        jj = sid + j * NS

        @pl.when(jj < half)
        def _():
            c = cid * half + jj
            pltpu.sync_copy(col_hbm.at[c], idx_v)
            pltpu.sync_copy(o_hbm.at[c], buf_v)
            pltpu.sync_copy(buf_v, acc_sh.at[idx_v], add=True)
        return carry

    lax.fori_loop(0, nj, body, 0)
    plsc.subcore_barrier()
    pltpu.sync_copy(acc_sh.at[pl.ds(r0, rows_per)],
                    p_hbm.at[cid, pl.ds(r0, rows_per)])


def _silu(x):
    return x / (1.0 + jnp.exp(-x))


def _dot(a, b):
    return jnp.dot(a, b, preferred_element_type=_f32)


def _edge_body(first, coords, refs):
    if first:
        (gcol, grow, freqs, w1c, w1r, b1, w2, b2, cw1, cb1, cw2,
         o_ref, ea_out) = refs
    else:
        (gcol, grow, ea_in, w1c, w1r, b1, w2, b2, cw1, cb1, cw2,
         o_ref) = refs
    A = gcol[:, :MSG]
    posc = gcol[:, MSG:MSG + 3]
    B = grow[:, :MSG]
    posr = grow[:, MSG:MSG + 3]
    cd = posr - posc
    radial = jnp.sum(cd * cd, axis=1, keepdims=True)
    if first:
        dist = jnp.sqrt(radial)
        ang = dist * freqs[...]
        ea = jnp.concatenate([jnp.sin(ang), jnp.cos(ang)], axis=1)
        ea_out[...] = ea
    else:
        ea = ea_in[...]
    z1 = A + B + _dot(ea, w1c[...]) + radial * w1r[...] + b1[...]
    m = _silu(z1)
    m = _silu(_dot(m, w2[...]) + b2[...])
    if coords:
        cw = _dot(_silu(_dot(m, cw1[...]) + cb1[...]), cw2[...])
        wcd = cd * cw
    else:
        wcd = jnp.zeros((m.shape[0], 3), _f32)
    ones = jnp.ones((m.shape[0], 1), _f32)
    pad = jnp.zeros((m.shape[0], TW - MSG - 4), _f32)
    o_ref[...] = jnp.concatenate([m, wcd, ones, pad], axis=1)


def _make_edge_call(first, coords):
    def body(*refs):
        _edge_body(first, coords, refs)

    grid = E // EB
    eblk = pl.BlockSpec((EB, TW), lambda i: (i, 0))
    eablk = pl.BlockSpec((EB, ED), lambda i: (i, 0))

    def w(shape):
        return pl.BlockSpec(shape, lambda i: tuple(0 for _ in shape))

    in_specs = [eblk, eblk]
    if first:
        in_specs.append(w((1, ED // 2)))
    else:
        in_specs.append(eablk)
    in_specs += [w((ED, MSG)), w((1, MSG)), w((1, MSG)), w((MSG, MSG)),
                 w((1, MSG)), w((MSG, MSG)), w((1, MSG)), w((MSG, 1))]
    out_shape = [jax.ShapeDtypeStruct((E, TW), _f32)]
    out_specs = [eblk]
    if first:
        out_shape.append(jax.ShapeDtypeStruct((E, ED), _f32))
        out_specs.append(eablk)
    return pl.pallas_call(
        body, grid=(grid,), in_specs=in_specs,
        out_specs=out_specs if len(out_specs) > 1 else out_specs[0],
        out_shape=out_shape if len(out_shape) > 1 else out_shape[0])


_edge_first = _make_edge_call(True, True)
_edge_mid = _make_edge_call(False, True)
_edge_last = _make_edge_call(False, False)


def _node_body(tables, refs):
    if tables:
        (h_ref, pos_ref, p_ref, w1h, w1m, nb1, w2, nb2, g, b, wa, wb,
         h_out, pos_out, tcol_out, trow_out) = refs
    else:
        (h_ref, pos_ref, p_ref, w1h, w1m, nb1, w2, nb2, g, b,
         h_out, pos_out) = refs
    S = p_ref[0] + p_ref[1]
    msum = S[:, :MSG]
    coordagg = S[:, MSG:MSG + 3]
    cnt = S[:, MSG + 3:MSG + 4]
    agg = msum / jnp.maximum(cnt, 1.0)
    h = h_ref[...]
    z = _dot(h, w1h[...]) + _dot(agg, w1m[...]) + nb1[...]
    h2 = h + _dot(_silu(z), w2[...]) + nb2[...]
    mu = jnp.mean(h2, axis=-1, keepdims=True)
    hc = h2 - mu
    var = jnp.mean(hc * hc, axis=-1, keepdims=True)
    hn = hc / jnp.sqrt(var + 1e-5) * g[...] + b[...]
    h_out[...] = hn
    pos_new = pos_ref[...] + coordagg
    pos_out[...] = pos_new
    if tables:
        padz = jnp.zeros((hn.shape[0], TW - MSG - 3), _f32)
        tcol_out[...] = jnp.concatenate([_dot(hn, wa[...]), pos_new, padz],
                                        axis=1)
        trow_out[...] = jnp.concatenate([_dot(hn, wb[...]), pos_new, padz],
                                        axis=1)


def _make_node_call(tables):
    def body(*refs):
        _node_body(tables, refs)

    grid = N // NB
    hblk = pl.BlockSpec((NB, D), lambda i: (i, 0))
    pblk = pl.BlockSpec((NB, 3), lambda i: (i, 0))
    tblk = pl.BlockSpec((NB, TW), lambda i: (i, 0))
    partblk = pl.BlockSpec((NC, NB, TW), lambda i: (0, i, 0))

    def w(shape):
        return pl.BlockSpec(shape, lambda i: tuple(0 for _ in shape))

    in_specs = [hblk, pblk, partblk, w((D, MSG)), w((MSG, MSG)), w((1, MSG)),
                w((MSG, D)), w((1, D)), w((1, D)), w((1, D))]
    out_shape = [jax.ShapeDtypeStruct((N, D), _f32),
                 jax.ShapeDtypeStruct((N, 3), _f32)]
    out_specs = [hblk, pblk]
    if tables:
        in_specs += [w((D, MSG)), w((D, MSG))]
        out_shape += [jax.ShapeDtypeStruct((N, TW), _f32),
                      jax.ShapeDtypeStruct((N, TW), _f32)]
        out_specs += [tblk, tblk]
    return pl.pallas_call(
        body, grid=(grid,), in_specs=in_specs,
        out_specs=out_specs, out_shape=out_shape)


_node_mid = _make_node_call(True)
_node_last = _make_node_call(False)


def _prologue_body(h_ref, pos_ref, ew, eb, wa, wb, h_out, tcol_out, trow_out):
    h0 = _dot(h_ref[...], ew[...]) + eb[...]
    h_out[...] = h0
    padz = jnp.zeros((h0.shape[0], TW - MSG - 3), _f32)
    pos = pos_ref[...]
    tcol_out[...] = jnp.concatenate([_dot(h0, wa[...]), pos, padz], axis=1)
    trow_out[...] = jnp.concatenate([_dot(h0, wb[...]), pos, padz], axis=1)


def _make_prologue():
    grid = N // NB
    hblk = pl.BlockSpec((NB, D), lambda i: (i, 0))
    pblk = pl.BlockSpec((NB, 3), lambda i: (i, 0))
    tblk = pl.BlockSpec((NB, TW), lambda i: (i, 0))

    def w(shape):
        return pl.BlockSpec(shape, lambda i: tuple(0 for _ in shape))

    return pl.pallas_call(
        _prologue_body, grid=(grid,),
        in_specs=[hblk, pblk, w((D, D)), w((1, D)), w((D, MSG)), w((D, MSG))],
        out_specs=[hblk, tblk, tblk],
        out_shape=[jax.ShapeDtypeStruct((N, D), _f32),
                   jax.ShapeDtypeStruct((N, TW), _f32),
                   jax.ShapeDtypeStruct((N, TW), _f32)])


_prologue = _make_prologue()


def kernel(h, pos, edge_index, embed_W, embed_b, msg_W1, msg_b1, msg_W2,
           msg_b2, coord_W1, coord_b1, coord_W2, node_W1, node_b1, node_W2,
           node_b2, ln_g, ln_b):
    row = edge_index[0].reshape(NCHUNK, CH)
    col = edge_index[1].reshape(NCHUNK, CH)
    freqs = jnp.exp(-math.log(10000.0)
                    * jnp.arange(0, ED, 2, dtype=_f32) / ED).reshape(1, ED // 2)
    zeros_tbl = jnp.zeros((N, TW), _f32)

    W1a = msg_W1[:, :D]
    W1b = msg_W1[:, D:2 * D]
    W1c = jnp.concatenate([msg_W1[:, 2 * D:2 * D + ED:2],
                           msg_W1[:, 2 * D + 1:2 * D + ED:2]], axis=1)
    w1r = msg_W1[:, 2 * D + ED].reshape(L, 1, MSG)
    b1 = msg_b1.reshape(L, 1, MSG)
    b2 = msg_b2.reshape(L, 1, MSG)
    cb1 = coord_b1.reshape(L - 1, 1, MSG)
    nW1h = node_W1[:, :D]
    nW1m = node_W1[:, D:]
    nb1 = node_b1.reshape(L, 1, MSG)
    nb2 = node_b2.reshape(L, 1, D)
    g = ln_g.reshape(L, 1, D)
    bb = ln_b.reshape(L, 1, D)

    hcur, tcol, trow = _prologue(h, pos, embed_W, embed_b.reshape(1, D),
                                 W1a[0], W1b[0])
    poscur = pos
    ea = None
    for l in range(L):
        gcol, grow = _gather_k(tcol, trow, col, row)
        gcol = gcol.reshape(E, TW)
        grow = grow.reshape(E, TW)
        if l == 0:
            O, ea = _edge_first(gcol, grow, freqs, W1c[l], w1r[l], b1[l],
                                msg_W2[l], b2[l], coord_W1[l], cb1[l],
                                coord_W2[l])
        elif l < L - 1:
            O = _edge_mid(gcol, grow, ea, W1c[l], w1r[l], b1[l],
                          msg_W2[l], b2[l], coord_W1[l], cb1[l], coord_W2[l])
        else:
            O = _edge_last(gcol, grow, ea, W1c[l], w1r[l], b1[l],
                           msg_W2[l], b2[l], coord_W1[0], cb1[0], coord_W2[0])
        P = _scatter_k(O.reshape(NCHUNK, CH, TW), col, zeros_tbl)
        if l < L - 1:
            hcur, poscur, tcol, trow = _node_mid(
                hcur, poscur, P, nW1h[l], nW1m[l], nb1[l], node_W2[l],
                nb2[l], g[l], bb[l], W1a[l + 1], W1b[l + 1])
        else:
            hcur, poscur = _node_last(
                hcur, poscur, P, nW1h[l], nW1m[l], nb1[l], node_W2[l],
                nb2[l], g[l], bb[l])
    return hcur, poscur


# speedup vs baseline: 3.7226x; 1.0003x over previous
import functools
import math

import jax
import jax.numpy as jnp
from jax import lax
from jax.experimental import pallas as pl
from jax.experimental.pallas import tpu as pltpu
from jax.experimental.pallas import tpu_sc as plsc

N = 10000
E = 160000
D = 128
MSG = 24
ED = 16
L = 6
TW = 32
CH = 128
NCHUNK = E // CH
NC, NS = 2, 16
NW = NC * NS
EB = 4000
NB = 2000

_f32 = jnp.float32

_mesh = plsc.VectorSubcoreMesh(
    core_axis_name="c", subcore_axis_name="s", num_cores=NC, num_subcores=NS)


@functools.partial(
    pl.kernel,
    out_type=(jax.ShapeDtypeStruct((E, TW), _f32),
              jax.ShapeDtypeStruct((E, TW), _f32)),
    mesh=_mesh,
    scratch_types=[pltpu.VMEM((CH,), jnp.int32),
                   pltpu.VMEM((CH,), jnp.int32),
                   pltpu.VMEM((CH, TW), _f32),
                   pltpu.VMEM((CH, TW), _f32),
                   pltpu.SemaphoreType.DMA,
                   pltpu.SemaphoreType.DMA],
    compiler_params=pltpu.CompilerParams(use_tc_tiling_on_sc=False),
)
def _gather_k(tcol_hbm, trow_hbm, col_hbm, row_hbm, gcol_hbm, grow_hbm,
              idxc_v, idxr_v, bufc_v, bufr_v, sem1, sem2):
    cid = lax.axis_index("c")
    sid = lax.axis_index("s")
    wid = sid * NC + cid
    nj = (NCHUNK + NW - 1) // NW

    def body(j, carry):
        c = wid + j * NW

        @pl.when(c < NCHUNK)
        def _():
            e0 = c * CH
            pltpu.sync_copy(col_hbm.at[pl.ds(e0, CH)], idxc_v)
            pltpu.sync_copy(row_hbm.at[pl.ds(e0, CH)], idxr_v)
            d1 = pltpu.async_copy(tcol_hbm.at[idxc_v], bufc_v, sem1)
            d2 = pltpu.async_copy(trow_hbm.at[idxr_v], bufr_v, sem2)
            d1.wait()
            d2.wait()
            pltpu.sync_copy(bufc_v, gcol_hbm.at[pl.ds(e0, CH)])
            pltpu.sync_copy(bufr_v, grow_hbm.at[pl.ds(e0, CH)])
        return carry

    lax.fori_loop(0, nj, body, 0)


@functools.partial(
    pl.kernel,
    out_type=jax.ShapeDtypeStruct((NC, N, TW), _f32),
    mesh=_mesh,
    scratch_types=[pltpu.VMEM((CH,), jnp.int32),
                   pltpu.VMEM((CH, TW), _f32),
                   pltpu.VMEM_SHARED((N, TW), _f32)],
    compiler_params=pltpu.CompilerParams(use_tc_tiling_on_sc=False),
)
def _scatter_k(o_hbm, col_hbm, zero_hbm, p_hbm, idx_v, buf_v, acc_sh):
    cid = lax.axis_index("c")
    sid = lax.axis_index("s")
    rows_per = N // NS
    r0 = sid * rows_per
    pltpu.sync_copy(zero_hbm.at[pl.ds(r0, rows_per)],
                    acc_sh.at[pl.ds(r0, rows_per)])
    plsc.subcore_barrier()
    half = NCHUNK // NC
    nj = (half + NS - 1) // NS

    def body(j, carry):
        jj = sid + j * NS

        @pl.when(jj < half)
        def _():
            c = cid * half + jj
            e0 = c * CH
            pltpu.sync_copy(col_hbm.at[pl.ds(e0, CH)], idx_v)
            pltpu.sync_copy(o_hbm.at[pl.ds(e0, CH)], buf_v)
            pltpu.sync_copy(buf_v, acc_sh.at[idx_v], add=True)
        return carry

    lax.fori_loop(0, nj, body, 0)
    plsc.subcore_barrier()
    pltpu.sync_copy(acc_sh.at[pl.ds(r0, rows_per)],
                    p_hbm.at[cid, pl.ds(r0, rows_per)])


def _silu(x):
    return x / (1.0 + jnp.exp(-x))


def _dot(a, b):
    return jnp.dot(a, b, preferred_element_type=_f32)


def _edge_body(first, coords, refs):
    if first:
        (gcol, grow, freqs, w1c, w1r, b1, w2, b2, cw1, cb1, cw2,
         o_ref, ea_out) = refs
    else:
        (gcol, grow, ea_in, w1c, w1r, b1, w2, b2, cw1, cb1, cw2,
         o_ref) = refs
    A = gcol[:, :MSG]
    posc = gcol[:, MSG:MSG + 3]
    B = grow[:, :MSG]
    posr = grow[:, MSG:MSG + 3]
    cd = posr - posc
    radial = jnp.sum(cd * cd, axis=1, keepdims=True)
    if first:
        dist = jnp.sqrt(radial)
        ang = dist * freqs[...]
        ea = jnp.concatenate([jnp.sin(ang), jnp.cos(ang)], axis=1)
        ea_out[...] = ea
    else:
        ea = ea_in[...]
    z1 = A + B + _dot(ea, w1c[...]) + radial * w1r[...] + b1[...]
    m = _silu(z1)
    m = _silu(_dot(m, w2[...]) + b2[...])
    if coords:
        cw = _dot(_silu(_dot(m, cw1[...]) + cb1[...]), cw2[...])
        wcd = cd * cw
    else:
        wcd = jnp.zeros((m.shape[0], 3), _f32)
    ones = jnp.ones((m.shape[0], 1), _f32)
    pad = jnp.zeros((m.shape[0], TW - MSG - 4), _f32)
    o_ref[...] = jnp.concatenate([m, wcd, ones, pad], axis=1)


def _make_edge_call(first, coords):
    def body(*refs):
        _edge_body(first, coords, refs)

    grid = E // EB
    eblk = pl.BlockSpec((EB, TW), lambda i: (i, 0))
    eablk = pl.BlockSpec((EB, ED), lambda i: (i, 0))

    def w(shape):
        return pl.BlockSpec(shape, lambda i: tuple(0 for _ in shape))

    in_specs = [eblk, eblk]
    if first:
        in_specs.append(w((1, ED // 2)))
    else:
        in_specs.append(eablk)
    in_specs += [w((ED, MSG)), w((1, MSG)), w((1, MSG)), w((MSG, MSG)),
                 w((1, MSG)), w((MSG, MSG)), w((1, MSG)), w((MSG, 1))]
    out_shape = [jax.ShapeDtypeStruct((E, TW), _f32)]
    out_specs = [eblk]
    if first:
        out_shape.append(jax.ShapeDtypeStruct((E, ED), _f32))
        out_specs.append(eablk)
    return pl.pallas_call(
        body, grid=(grid,), in_specs=in_specs,
        out_specs=out_specs if len(out_specs) > 1 else out_specs[0],
        out_shape=out_shape if len(out_shape) > 1 else out_shape[0])


_edge_first = _make_edge_call(True, True)
_edge_mid = _make_edge_call(False, True)
_edge_last = _make_edge_call(False, False)


def _node_body(tables, refs):
    if tables:
        (h_ref, pos_ref, p_ref, w1h, w1m, nb1, w2, nb2, g, b, wa, wb,
         h_out, pos_out, tcol_out, trow_out) = refs
    else:
        (h_ref, pos_ref, p_ref, w1h, w1m, nb1, w2, nb2, g, b,
         h_out, pos_out) = refs
    S = p_ref[0] + p_ref[1]
    msum = S[:, :MSG]
    coordagg = S[:, MSG:MSG + 3]
    cnt = S[:, MSG + 3:MSG + 4]
    agg = msum / jnp.maximum(cnt, 1.0)
    h = h_ref[...]
    z = _dot(h, w1h[...]) + _dot(agg, w1m[...]) + nb1[...]
    h2 = h + _dot(_silu(z), w2[...]) + nb2[...]
    mu = jnp.mean(h2, axis=-1, keepdims=True)
    hc = h2 - mu
    var = jnp.mean(hc * hc, axis=-1, keepdims=True)
    hn = hc / jnp.sqrt(var + 1e-5) * g[...] + b[...]
    h_out[...] = hn
    pos_new = pos_ref[...] + coordagg
    pos_out[...] = pos_new
    if tables:
        padz = jnp.zeros((hn.shape[0], TW - MSG - 3), _f32)
        tcol_out[...] = jnp.concatenate([_dot(hn, wa[...]), pos_new, padz],
                                        axis=1)
        trow_out[...] = jnp.concatenate([_dot(hn, wb[...]), pos_new, padz],
                                        axis=1)


def _make_node_call(tables):
    def body(*refs):
        _node_body(tables, refs)

    grid = N // NB
    hblk = pl.BlockSpec((NB, D), lambda i: (i, 0))
    pblk = pl.BlockSpec((NB, 3), lambda i: (i, 0))
    tblk = pl.BlockSpec((NB, TW), lambda i: (i, 0))
    partblk = pl.BlockSpec((NC, NB, TW), lambda i: (0, i, 0))

    def w(shape):
        return pl.BlockSpec(shape, lambda i: tuple(0 for _ in shape))

    in_specs = [hblk, pblk, partblk, w((D, MSG)), w((MSG, MSG)), w((1, MSG)),
                w((MSG, D)), w((1, D)), w((1, D)), w((1, D))]
    out_shape = [jax.ShapeDtypeStruct((N, D), _f32),
                 jax.ShapeDtypeStruct((N, 3), _f32)]
    out_specs = [hblk, pblk]
    if tables:
        in_specs += [w((D, MSG)), w((D, MSG))]
        out_shape += [jax.ShapeDtypeStruct((N, TW), _f32),
                      jax.ShapeDtypeStruct((N, TW), _f32)]
        out_specs += [tblk, tblk]
    return pl.pallas_call(
        body, grid=(grid,), in_specs=in_specs,
        out_specs=out_specs, out_shape=out_shape)


_node_mid = _make_node_call(True)
_node_last = _make_node_call(False)


def _prologue_body(h_ref, pos_ref, ew, eb, wa, wb, h_out, tcol_out, trow_out):
    h0 = _dot(h_ref[...], ew[...]) + eb[...]
    h_out[...] = h0
    padz = jnp.zeros((h0.shape[0], TW - MSG - 3), _f32)
    pos = pos_ref[...]
    tcol_out[...] = jnp.concatenate([_dot(h0, wa[...]), pos, padz], axis=1)
    trow_out[...] = jnp.concatenate([_dot(h0, wb[...]), pos, padz], axis=1)


def _make_prologue():
    grid = N // NB
    hblk = pl.BlockSpec((NB, D), lambda i: (i, 0))
    pblk = pl.BlockSpec((NB, 3), lambda i: (i, 0))
    tblk = pl.BlockSpec((NB, TW), lambda i: (i, 0))

    def w(shape):
        return pl.BlockSpec(shape, lambda i: tuple(0 for _ in shape))

    return pl.pallas_call(
        _prologue_body, grid=(grid,),
        in_specs=[hblk, pblk, w((D, D)), w((1, D)), w((D, MSG)), w((D, MSG))],
        out_specs=[hblk, tblk, tblk],
        out_shape=[jax.ShapeDtypeStruct((N, D), _f32),
                   jax.ShapeDtypeStruct((N, TW), _f32),
                   jax.ShapeDtypeStruct((N, TW), _f32)])


_prologue = _make_prologue()


def kernel(h, pos, edge_index, embed_W, embed_b, msg_W1, msg_b1, msg_W2,
           msg_b2, coord_W1, coord_b1, coord_W2, node_W1, node_b1, node_W2,
           node_b2, ln_g, ln_b):
    row = edge_index[0]
    col = edge_index[1]
    freqs = jnp.exp(-math.log(10000.0)
                    * jnp.arange(0, ED, 2, dtype=_f32) / ED).reshape(1, ED // 2)
    zeros_tbl = jnp.zeros((N, TW), _f32)

    W1a = msg_W1[:, :D]
    W1b = msg_W1[:, D:2 * D]
    W1c = jnp.concatenate([msg_W1[:, 2 * D:2 * D + ED:2],
                           msg_W1[:, 2 * D + 1:2 * D + ED:2]], axis=1)
    w1r = msg_W1[:, 2 * D + ED].reshape(L, 1, MSG)
    b1 = msg_b1.reshape(L, 1, MSG)
    b2 = msg_b2.reshape(L, 1, MSG)
    cb1 = coord_b1.reshape(L - 1, 1, MSG)
    nW1h = node_W1[:, :D]
    nW1m = node_W1[:, D:]
    nb1 = node_b1.reshape(L, 1, MSG)
    nb2 = node_b2.reshape(L, 1, D)
    g = ln_g.reshape(L, 1, D)
    bb = ln_b.reshape(L, 1, D)

    hcur, tcol, trow = _prologue(h, pos, embed_W, embed_b.reshape(1, D),
                                 W1a[0], W1b[0])
    poscur = pos
    ea = None
    for l in range(L):
        gcol, grow = _gather_k(tcol, trow, col, row)
        if l == 0:
            O, ea = _edge_first(gcol, grow, freqs, W1c[l], w1r[l], b1[l],
                                msg_W2[l], b2[l], coord_W1[l], cb1[l],
                                coord_W2[l])
        elif l < L - 1:
            O = _edge_mid(gcol, grow, ea, W1c[l], w1r[l], b1[l],
                          msg_W2[l], b2[l], coord_W1[l], cb1[l], coord_W2[l])
        else:
            O = _edge_last(gcol, grow, ea, W1c[l], w1r[l], b1[l],
                           msg_W2[l], b2[l], coord_W1[0], cb1[0], coord_W2[0])
        P = _scatter_k(O, col, zeros_tbl)
        if l < L - 1:
            hcur, poscur, tcol, trow = _node_mid(
                hcur, poscur, P, nW1h[l], nW1m[l], nb1[l], node_W2[l],
                nb2[l], g[l], bb[l], W1a[l + 1], W1b[l + 1])
        else:
            hcur, poscur = _node_last(
                hcur, poscur, P, nW1h[l], nW1m[l], nb1[l], node_W2[l],
                nb2[l], g[l], bb[l])
    return hcur, poscur


# speedup vs baseline: 7.1301x; 1.9154x over previous
import functools
import math

import jax
import jax.numpy as jnp
from jax import lax
from jax.experimental import pallas as pl
from jax.experimental.pallas import tpu as pltpu
from jax.experimental.pallas import tpu_sc as plsc

N = 10000
E = 160000
D = 128
MSG = 24
ED = 16
L = 6
TW = 32
CH = 128
NCHUNK = E // CH
NC, NS = 2, 16
NW = NC * NS
EB = 4000
NB = 2000

_f32 = jnp.float32

_mesh = plsc.VectorSubcoreMesh(
    core_axis_name="c", subcore_axis_name="s", num_cores=NC, num_subcores=NS)


@functools.partial(
    pl.kernel,
    out_type=(jax.ShapeDtypeStruct((E, TW), _f32),
              jax.ShapeDtypeStruct((E, TW), _f32)),
    mesh=_mesh,
    scratch_types=[pltpu.VMEM((CH,), jnp.int32),
                   pltpu.VMEM((CH,), jnp.int32),
                   pltpu.VMEM((CH, TW), _f32),
                   pltpu.VMEM((CH, TW), _f32),
                   pltpu.SemaphoreType.DMA,
                   pltpu.SemaphoreType.DMA],
    compiler_params=pltpu.CompilerParams(use_tc_tiling_on_sc=False),
)
def _gather_k(tcol_hbm, trow_hbm, col_hbm, row_hbm, gcol_hbm, grow_hbm,
              idxc_v, idxr_v, bufc_v, bufr_v, sem1, sem2):
    cid = lax.axis_index("c")
    sid = lax.axis_index("s")
    wid = sid * NC + cid
    nj = (NCHUNK + NW - 1) // NW

    def body(j, carry):
        c = wid + j * NW

        @pl.when(c < NCHUNK)
        def _():
            e0 = c * CH
            pltpu.sync_copy(col_hbm.at[pl.ds(e0, CH)], idxc_v)
            pltpu.sync_copy(row_hbm.at[pl.ds(e0, CH)], idxr_v)
            d1 = pltpu.async_copy(tcol_hbm.at[idxc_v], bufc_v, sem1)
            d2 = pltpu.async_copy(trow_hbm.at[idxr_v], bufr_v, sem2)
            d1.wait()
            d2.wait()
            pltpu.sync_copy(bufc_v, gcol_hbm.at[pl.ds(e0, CH)])
            pltpu.sync_copy(bufr_v, grow_hbm.at[pl.ds(e0, CH)])
        return carry

    lax.fori_loop(0, nj, body, 0)


@functools.partial(
    pl.kernel,
    out_type=jax.ShapeDtypeStruct((NC, N, TW), _f32),
    mesh=_mesh,
    scratch_types=[pltpu.VMEM((CH,), jnp.int32),
                   pltpu.VMEM((CH, TW), _f32),
                   pltpu.VMEM_SHARED((N, TW), _f32)],
    compiler_params=pltpu.CompilerParams(use_tc_tiling_on_sc=False),
)
def _scatter_k(o_hbm, col_hbm, zero_hbm, p_hbm, idx_v, buf_v, acc_sh):
    cid = lax.axis_index("c")
    sid = lax.axis_index("s")
    rows_per = N // NS
    r0 = sid * rows_per
    pltpu.sync_copy(zero_hbm.at[pl.ds(r0, rows_per)],
                    acc_sh.at[pl.ds(r0, rows_per)])
    plsc.subcore_barrier()
    half = NCHUNK // NC
    nj = (half + NS - 1) // NS

    def body(j, carry):
        jj = sid + j * NS

        @pl.when(jj < half)
        def _():
            c = cid * half + jj
            e0 = c * CH
            pltpu.sync_copy(col_hbm.at[pl.ds(e0, CH)], idx_v)
            pltpu.sync_copy(o_hbm.at[pl.ds(e0, CH)], buf_v)
            pltpu.sync_copy(buf_v, acc_sh.at[idx_v], add=True)
        return carry

    lax.fori_loop(0, nj, body, 0)
    plsc.subcore_barrier()
    pltpu.sync_copy(acc_sh.at[pl.ds(r0, rows_per)],
                    p_hbm.at[cid, pl.ds(r0, rows_per)])


def _silu(x):
    return x / (1.0 + jnp.exp(-x))


def _dot(a, b):
    return jnp.dot(a, b, preferred_element_type=_f32)


def _edge_body(first, coords, refs):
    if first:
        (gcol, grow, mS, mD, Mrad, Wr4, b1bc, W2bd, b2bc, C1bd, cb1bc, C2p,
         E4, onesP, Wc64, F4, mSin, o_ref, ea_out) = refs
    else:
        (gcol, grow, ea_in, mS, mD, Mrad, Wr4, b1bc, W2bd, b2bc, C1bd, cb1bc,
         C2p, E4, onesP, Wc64, o_ref) = refs
    S = gcol[...] + grow[...]
    Dd = grow[...] - gcol[...]
    cdP = Dd * mD[...]
    R4 = _dot(cdP * cdP, Mrad[...])
    if first:
        ang = _dot(jnp.sqrt(R4), F4[...])
        eaP = jnp.where(mSin[...] > 0.5, jnp.sin(ang), jnp.cos(ang))
        ea_out[...] = eaP
    else:
        eaP = ea_in[...]
    z1 = S * mS[...] + _dot(R4, Wr4[...]) + _dot(eaP, Wc64[...]) + b1bc[...]
    m1 = _silu(z1)
    m2 = _silu(_dot(m1, W2bd[...]) + b2bc[...])
    if coords:
        t = _silu(_dot(m2, C1bd[...]) + cb1bc[...])
        cw4 = _dot(t, C2p[...])
        o_ref[...] = m2 + cdP * _dot(cw4, E4[...]) + onesP[...]
    else:
        o_ref[...] = m2 + onesP[...]


def _make_edge_call(first, coords):
    def body(*refs):
        _edge_body(first, coords, refs)

    grid = E // EB
    EBR = EB // 4
    eblk = pl.BlockSpec((EBR, 128), lambda i: (i, 0))
    eablk = pl.BlockSpec((EBR, 64), lambda i: (i, 0))

    def w(shape):
        return pl.BlockSpec(shape, lambda i: tuple(0 for _ in shape))

    in_specs = [eblk, eblk]
    if not first:
        in_specs.append(eablk)
    in_specs += [w((1, 128)), w((1, 128)), w((128, 4)), w((4, 128)),
                 w((1, 128)), w((128, 128)), w((1, 128)), w((128, 128)),
                 w((1, 128)), w((128, 4)), w((4, 128)), w((1, 128)),
                 w((64, 128))]
    if first:
        in_specs += [w((4, 64)), w((1, 64))]
    out_shape = [jax.ShapeDtypeStruct((E // 4, 128), _f32)]
    out_specs = [eblk]
    if first:
        out_shape.append(jax.ShapeDtypeStruct((E // 4, 64), _f32))
        out_specs.append(eablk)
    return pl.pallas_call(
        body, grid=(grid,), in_specs=in_specs,
        out_specs=out_specs if len(out_specs) > 1 else out_specs[0],
        out_shape=out_shape if len(out_shape) > 1 else out_shape[0])


_edge_first = _make_edge_call(True, True)
_edge_mid = _make_edge_call(False, True)
_edge_last = _make_edge_call(False, False)


def _node_body(tables, refs):
    if tables:
        (h_ref, pos_ref, p_ref, w1h, w1m, nb1, w2, nb2, g, b, wa, wb,
         h_out, pos_out, tcol_out, trow_out) = refs
    else:
        (h_ref, pos_ref, p_ref, w1h, w1m, nb1, w2, nb2, g, b,
         h_out, pos_out) = refs
    S = p_ref[0] + p_ref[1]
    msum = S[:, :MSG]
    coordagg = S[:, MSG:MSG + 3]
    cnt = S[:, MSG + 3:MSG + 4]
    agg = msum / jnp.maximum(cnt, 1.0)
    h = h_ref[...]
    z = _dot(h, w1h[...]) + _dot(agg, w1m[...]) + nb1[...]
    h2 = h + _dot(_silu(z), w2[...]) + nb2[...]
    mu = jnp.mean(h2, axis=-1, keepdims=True)
    hc = h2 - mu
    var = jnp.mean(hc * hc, axis=-1, keepdims=True)
    hn = hc / jnp.sqrt(var + 1e-5) * g[...] + b[...]
    h_out[...] = hn
    pos_new = pos_ref[...] + coordagg
    pos_out[...] = pos_new
    if tables:
        padz = jnp.zeros((hn.shape[0], TW - MSG - 3), _f32)
        tcol_out[...] = jnp.concatenate([_dot(hn, wa[...]), pos_new, padz],
                                        axis=1)
        trow_out[...] = jnp.concatenate([_dot(hn, wb[...]), pos_new, padz],
                                        axis=1)


def _make_node_call(tables):
    def body(*refs):
        _node_body(tables, refs)

    grid = N // NB
    hblk = pl.BlockSpec((NB, D), lambda i: (i, 0))
    pblk = pl.BlockSpec((NB, 3), lambda i: (i, 0))
    tblk = pl.BlockSpec((NB, TW), lambda i: (i, 0))
    partblk = pl.BlockSpec((NC, NB, TW), lambda i: (0, i, 0))

    def w(shape):
        return pl.BlockSpec(shape, lambda i: tuple(0 for _ in shape))

    in_specs = [hblk, pblk, partblk, w((D, MSG)), w((MSG, MSG)), w((1, MSG)),
                w((MSG, D)), w((1, D)), w((1, D)), w((1, D))]
    out_shape = [jax.ShapeDtypeStruct((N, D), _f32),
                 jax.ShapeDtypeStruct((N, 3), _f32)]
    out_specs = [hblk, pblk]
    if tables:
        in_specs += [w((D, MSG)), w((D, MSG))]
        out_shape += [jax.ShapeDtypeStruct((N, TW), _f32),
                      jax.ShapeDtypeStruct((N, TW), _f32)]
        out_specs += [tblk, tblk]
    return pl.pallas_call(
        body, grid=(grid,), in_specs=in_specs,
        out_specs=out_specs, out_shape=out_shape)


_node_mid = _make_node_call(True)
_node_last = _make_node_call(False)


def _prologue_body(h_ref, pos_ref, ew, eb, wa, wb, h_out, tcol_out, trow_out):
    h0 = _dot(h_ref[...], ew[...]) + eb[...]
    h_out[...] = h0
    padz = jnp.zeros((h0.shape[0], TW - MSG - 3), _f32)
    pos = pos_ref[...]
    tcol_out[...] = jnp.concatenate([_dot(h0, wa[...]), pos, padz], axis=1)
    trow_out[...] = jnp.concatenate([_dot(h0, wb[...]), pos, padz], axis=1)


def _make_prologue():
    grid = N // NB
    hblk = pl.BlockSpec((NB, D), lambda i: (i, 0))
    pblk = pl.BlockSpec((NB, 3), lambda i: (i, 0))
    tblk = pl.BlockSpec((NB, TW), lambda i: (i, 0))

    def w(shape):
        return pl.BlockSpec(shape, lambda i: tuple(0 for _ in shape))

    return pl.pallas_call(
        _prologue_body, grid=(grid,),
        in_specs=[hblk, pblk, w((D, D)), w((1, D)), w((D, MSG)), w((D, MSG))],
        out_specs=[hblk, tblk, tblk],
        out_shape=[jax.ShapeDtypeStruct((N, D), _f32),
                   jax.ShapeDtypeStruct((N, TW), _f32),
                   jax.ShapeDtypeStruct((N, TW), _f32)])


_prologue = _make_prologue()


def kernel(h, pos, edge_index, embed_W, embed_b, msg_W1, msg_b1, msg_W2,
           msg_b2, coord_W1, coord_b1, coord_W2, node_W1, node_b1, node_W2,
           node_b2, ln_g, ln_b):
    row = edge_index[0]
    col = edge_index[1]
    freqs = jnp.exp(-math.log(10000.0)
                    * jnp.arange(0, ED, 2, dtype=_f32) / ED)
    zeros_tbl = jnp.zeros((N, TW), _f32)

    W1a = msg_W1[:, :D]
    W1b = msg_W1[:, D:2 * D]
    W1c = jnp.concatenate([msg_W1[:, 2 * D:2 * D + ED:2],
                           msg_W1[:, 2 * D + 1:2 * D + ED:2]], axis=1)
    w1r = msg_W1[:, 2 * D + ED]

    c128 = jnp.arange(128)
    m32 = c128 % 32
    grp = c128 // 32
    in_msg = m32 < MSG
    in_cd = (m32 >= MSG) & (m32 < MSG + 3)
    oh4 = (jnp.arange(4)[:, None] == grp[None, :]).astype(_f32)
    mS = in_msg.astype(_f32)[None]
    mD = in_cd.astype(_f32)[None]
    Mrad = (oh4.T * in_cd[:, None].astype(_f32))
    E4 = oh4 * mD
    onesP = (m32 == MSG + 3).astype(_f32)[None]
    mclip = jnp.minimum(m32, MSG - 1)
    c64 = jnp.arange(64)
    r16 = c64 % 16
    g16 = c64 // 16
    oh16 = (jnp.arange(4)[:, None] == g16[None, :]).astype(_f32)
    F4 = oh16 * freqs[c64 % 8][None, :]
    mSin = ((c64 % 16) < 8).astype(_f32)[None]
    bd_mask = ((grp[:, None] == grp[None, :]) & in_msg[:, None]
               & in_msg[None, :]).astype(_f32)

    def bc24(v):
        return (jnp.where(in_msg, v[mclip], 0.0)).reshape(1, 128)

    def bd(Wsmall):
        return Wsmall[mclip][:, mclip] * bd_mask

    Wr4_l = [oh4 * bc24(w1r[l]) for l in range(L)]
    b1bc_l = [bc24(msg_b1[l]) for l in range(L)]
    b2bc_l = [bc24(msg_b2[l]) for l in range(L)]
    W2bd_l = [bd(msg_W2[l]) for l in range(L)]
    C1bd_l = [bd(coord_W1[l]) for l in range(L - 1)]
    cb1bc_l = [bc24(coord_b1[l]) for l in range(L - 1)]
    C2p_l = [(oh4.T * jnp.where(in_msg, coord_W2[l][mclip, 0], 0.0)[:, None])
             for l in range(L - 1)]
    wc_mask = (g16[:, None] == grp[None, :]).astype(_f32) * mS
    Wc64_l = [W1c[l][r16][:, mclip] * wc_mask for l in range(L)]
    nW1h = node_W1[:, :D]
    nW1m = node_W1[:, D:]
    nb1 = node_b1.reshape(L, 1, MSG)
    nb2 = node_b2.reshape(L, 1, D)
    g = ln_g.reshape(L, 1, D)
    bb = ln_b.reshape(L, 1, D)

    hcur, tcol, trow = _prologue(h, pos, embed_W, embed_b.reshape(1, D),
                                 W1a[0], W1b[0])
    poscur = pos
    ea = None
    for l in range(L):
        gcol, grow = _gather_k(tcol, trow, col, row)
        gcolP = gcol.reshape(E // 4, 128)
        growP = grow.reshape(E // 4, 128)
        if l == 0:
            OP, ea = _edge_first(gcolP, growP, mS, mD, Mrad, Wr4_l[l],
                                 b1bc_l[l], W2bd_l[l], b2bc_l[l], C1bd_l[l],
                                 cb1bc_l[l], C2p_l[l], E4, onesP, Wc64_l[l],
                                 F4, mSin)
        elif l < L - 1:
            OP = _edge_mid(gcolP, growP, ea, mS, mD, Mrad, Wr4_l[l],
                           b1bc_l[l], W2bd_l[l], b2bc_l[l], C1bd_l[l],
                           cb1bc_l[l], C2p_l[l], E4, onesP, Wc64_l[l])
        else:
            OP = _edge_last(gcolP, growP, ea, mS, mD, Mrad, Wr4_l[l],
                            b1bc_l[l], W2bd_l[l], b2bc_l[l], C1bd_l[0],
                            cb1bc_l[0], C2p_l[0], E4, onesP, Wc64_l[l])
        P = _scatter_k(OP.reshape(E, TW), col, zeros_tbl)
        if l < L - 1:
            hcur, poscur, tcol, trow = _node_mid(
                hcur, poscur, P, nW1h[l], nW1m[l], nb1[l], node_W2[l],
                nb2[l], g[l], bb[l], W1a[l + 1], W1b[l + 1])
        else:
            hcur, poscur = _node_last(
                hcur, poscur, P, nW1h[l], nW1m[l], nb1[l], node_W2[l],
                nb2[l], g[l], bb[l])
    return hcur, poscur


# speedup vs baseline: 8.5361x; 1.1972x over previous
import functools
import math

import jax
import jax.numpy as jnp
from jax import lax
from jax.experimental import pallas as pl
from jax.experimental.pallas import tpu as pltpu
from jax.experimental.pallas import tpu_sc as plsc

N = 10000
E = 160000
D = 128
MSG = 24
ED = 16
L = 6
TW = 32
CH = 128
NC, NS = 2, 16
NW = NC * NS
E2 = 163840
NCH2 = E2 // CH
CPW = NCH2 // NW
SK = 4
EB = 4096
NB = 2000

_f32 = jnp.float32

_mesh = plsc.VectorSubcoreMesh(
    core_axis_name="c", subcore_axis_name="s", num_cores=NC, num_subcores=NS)


NSUP = CPW // SK


@functools.partial(
    pl.kernel,
    out_type=(jax.ShapeDtypeStruct((E2, TW), _f32),
              jax.ShapeDtypeStruct((E2, TW), _f32)),
    mesh=_mesh,
    scratch_types=[pltpu.VMEM((CPW, CH), jnp.int32),
                   pltpu.VMEM((CPW, CH), jnp.int32),
                   pltpu.VMEM((2, SK * CH, TW), _f32),
                   pltpu.VMEM((2, SK * CH, TW), _f32),
                   pltpu.SemaphoreType.DMA,
                   pltpu.SemaphoreType.DMA,
                   pltpu.SemaphoreType.DMA],
    compiler_params=pltpu.CompilerParams(use_tc_tiling_on_sc=False),
)
def _gather_k(tcol_hbm, trow_hbm, col2_hbm, row2_hbm, gcol_hbm, grow_hbm,
              idxc_v, idxr_v, bufc_v, bufr_v, gsem, wsemc, wsemr):
    cid = lax.axis_index("c")
    sid = lax.axis_index("s")
    wid = sid * NC + cid
    c0 = wid * CPW
    e0 = c0 * CH
    pltpu.sync_copy(col2_hbm.at[pl.ds(c0, CPW)], idxc_v)
    pltpu.sync_copy(row2_hbm.at[pl.ds(c0, CPW)], idxr_v)

    def body(s, carry):
        b = s % 2
        eo = e0 + s * SK * CH

        @pl.when(s >= 2)
        def _():
            pltpu.make_async_copy(
                bufc_v.at[b], gcol_hbm.at[pl.ds(eo, SK * CH)], wsemc).wait()
            pltpu.make_async_copy(
                bufr_v.at[b], grow_hbm.at[pl.ds(eo, SK * CH)], wsemr).wait()

        ds_ = []
        for k in range(SK):
            c = s * SK + k
            ds_.append(pltpu.async_copy(
                tcol_hbm.at[idxc_v.at[c]],
                bufc_v.at[b, pl.ds(k * CH, CH)], gsem))
            ds_.append(pltpu.async_copy(
                trow_hbm.at[idxr_v.at[c]],
                bufr_v.at[b, pl.ds(k * CH, CH)], gsem))
        for d in ds_:
            d.wait()
        pltpu.async_copy(bufc_v.at[b], gcol_hbm.at[pl.ds(eo, SK * CH)], wsemc)
        pltpu.async_copy(bufr_v.at[b], grow_hbm.at[pl.ds(eo, SK * CH)], wsemr)
        return carry

    lax.fori_loop(0, NSUP, body, 0)
    for s in (NSUP - 2, NSUP - 1):
        eo = e0 + s * SK * CH
        pltpu.make_async_copy(
            bufc_v.at[s % 2], gcol_hbm.at[pl.ds(eo, SK * CH)], wsemc).wait()
        pltpu.make_async_copy(
            bufr_v.at[s % 2], grow_hbm.at[pl.ds(eo, SK * CH)], wsemr).wait()


@functools.partial(
    pl.kernel,
    out_type=jax.ShapeDtypeStruct((NC, N, TW), _f32),
    mesh=_mesh,
    scratch_types=[pltpu.VMEM((CPW, CH), jnp.int32),
                   pltpu.VMEM((2, SK * CH, TW), _f32),
                   pltpu.VMEM_SHARED((N + 8, TW), _f32),
                   pltpu.SemaphoreType.DMA],
    compiler_params=pltpu.CompilerParams(use_tc_tiling_on_sc=False),
)
def _scatter_k(o_hbm, col2_hbm, zero_hbm, p_hbm, idx2_v, obuf_v, acc_sh, lsem):
    cid = lax.axis_index("c")
    sid = lax.axis_index("s")
    rows_per = N // NS
    r0 = sid * rows_per
    pltpu.sync_copy(zero_hbm.at[pl.ds(r0, rows_per)],
                    acc_sh.at[pl.ds(r0, rows_per)])
    c0 = cid * (NCH2 // NC) + sid * CPW
    e0 = c0 * CH
    pltpu.sync_copy(col2_hbm.at[pl.ds(c0, CPW)], idx2_v)
    pltpu.async_copy(o_hbm.at[pl.ds(e0, SK * CH)], obuf_v.at[0], lsem)
    plsc.subcore_barrier()

    def body(s, carry):
        b = s % 2
        pltpu.make_async_copy(
            o_hbm.at[pl.ds(e0, SK * CH)], obuf_v.at[b], lsem).wait()

        @pl.when(s + 1 < NSUP)
        def _():
            pltpu.async_copy(o_hbm.at[pl.ds(e0 + (s + 1) * SK * CH, SK * CH)],
                             obuf_v.at[(s + 1) % 2], lsem)
        for k in range(SK):
            c = s * SK + k
            pltpu.sync_copy(obuf_v.at[b, pl.ds(k * CH, CH)],
                            acc_sh.at[idx2_v.at[c]], add=True)
        return carry

    lax.fori_loop(0, NSUP, body, 0)
    plsc.subcore_barrier()
    pltpu.sync_copy(acc_sh.at[pl.ds(r0, rows_per)],
                    p_hbm.at[cid, pl.ds(r0, rows_per)])


def _silu(x):
    return x / (1.0 + jnp.exp(-x))


def _dot(a, b):
    return jnp.dot(a, b, preferred_element_type=_f32)


def _edge_body(first, coords, refs):
    if first:
        (gcol, grow, mS, mD, Mrad, Wr4, b1bc, W2bd, b2bc, C1bd, cb1bc, C2p,
         E4, onesP, Wc64, F4, cosOff, o_ref, ea_out) = refs
    else:
        (gcol, grow, ea_in, mS, mD, Mrad, Wr4, b1bc, W2bd, b2bc, C1bd, cb1bc,
         C2p, E4, onesP, Wc64, o_ref) = refs
    S = gcol[...] + grow[...]
    Dd = grow[...] - gcol[...]
    cdP = Dd * mD[...]
    R4 = _dot(cdP * cdP, Mrad[...])
    if first:
        ang = _dot(jnp.sqrt(R4), F4[...]) + cosOff[...]
        eaP = jnp.sin(ang)
        ea_out[...] = eaP
    else:
        eaP = ea_in[...]
    z1 = S * mS[...] + _dot(R4, Wr4[...]) + _dot(eaP, Wc64[...]) + b1bc[...]
    m1 = _silu(z1)
    m2 = _silu(_dot(m1, W2bd[...]) + b2bc[...])
    if coords:
        t = _silu(_dot(m2, C1bd[...]) + cb1bc[...])
        cw4 = _dot(t, C2p[...])
        o_ref[...] = m2 + cdP * _dot(cw4, E4[...]) + onesP[...]
    else:
        o_ref[...] = m2 + onesP[...]


def _make_edge_call(first, coords):
    def body(*refs):
        _edge_body(first, coords, refs)

    grid = E2 // EB
    EBR = EB // 4
    eblk = pl.BlockSpec((EBR, 128), lambda i: (i, 0))
    eablk = pl.BlockSpec((EBR, 64), lambda i: (i, 0))

    def w(shape):
        return pl.BlockSpec(shape, lambda i: tuple(0 for _ in shape))

    in_specs = [eblk, eblk]
    if not first:
        in_specs.append(eablk)
    in_specs += [w((1, 128)), w((1, 128)), w((128, 4)), w((4, 128)),
                 w((1, 128)), w((128, 128)), w((1, 128)), w((128, 128)),
                 w((1, 128)), w((128, 4)), w((4, 128)), w((1, 128)),
                 w((64, 128))]
    if first:
        in_specs += [w((4, 64)), w((1, 64))]
    out_shape = [jax.ShapeDtypeStruct((E2 // 4, 128), _f32)]
    out_specs = [eblk]
    if first:
        out_shape.append(jax.ShapeDtypeStruct((E2 // 4, 64), _f32))
        out_specs.append(eablk)
    return pl.pallas_call(
        body, grid=(grid,), in_specs=in_specs,
        out_specs=out_specs if len(out_specs) > 1 else out_specs[0],
        out_shape=out_shape if len(out_shape) > 1 else out_shape[0])


_edge_first = _make_edge_call(True, True)
_edge_mid = _make_edge_call(False, True)
_edge_last = _make_edge_call(False, False)


def _node_body(tables, refs):
    if tables:
        (h_ref, pos_ref, p_ref, w1h, w1m, nb1, w2, nb2, g, b, wa, wb,
         h_out, pos_out, tcol_out, trow_out) = refs
    else:
        (h_ref, pos_ref, p_ref, w1h, w1m, nb1, w2, nb2, g, b,
         h_out, pos_out) = refs
    S = p_ref[0] + p_ref[1]
    msum = S[:, :MSG]
    coordagg = S[:, MSG:MSG + 3]
    cnt = S[:, MSG + 3:MSG + 4]
    agg = msum / jnp.maximum(cnt, 1.0)
    h = h_ref[...]
    z = _dot(h, w1h[...]) + _dot(agg, w1m[...]) + nb1[...]
    h2 = h + _dot(_silu(z), w2[...]) + nb2[...]
    mu = jnp.mean(h2, axis=-1, keepdims=True)
    hc = h2 - mu
    var = jnp.mean(hc * hc, axis=-1, keepdims=True)
    hn = hc / jnp.sqrt(var + 1e-5) * g[...] + b[...]
    h_out[...] = hn
    pos_new = pos_ref[...] + coordagg
    pos_out[...] = pos_new
    if tables:
        padz = jnp.zeros((hn.shape[0], TW - MSG - 3), _f32)
        tcol_out[...] = jnp.concatenate([_dot(hn, wa[...]), pos_new, padz],
                                        axis=1)
        trow_out[...] = jnp.concatenate([_dot(hn, wb[...]), pos_new, padz],
                                        axis=1)


def _make_node_call(tables):
    def body(*refs):
        _node_body(tables, refs)

    grid = N // NB
    hblk = pl.BlockSpec((NB, D), lambda i: (i, 0))
    pblk = pl.BlockSpec((NB, 3), lambda i: (i, 0))
    tblk = pl.BlockSpec((NB, TW), lambda i: (i, 0))
    partblk = pl.BlockSpec((NC, NB, TW), lambda i: (0, i, 0))

    def w(shape):
        return pl.BlockSpec(shape, lambda i: tuple(0 for _ in shape))

    in_specs = [hblk, pblk, partblk, w((D, MSG)), w((MSG, MSG)), w((1, MSG)),
                w((MSG, D)), w((1, D)), w((1, D)), w((1, D))]
    out_shape = [jax.ShapeDtypeStruct((N, D), _f32),
                 jax.ShapeDtypeStruct((N, 3), _f32)]
    out_specs = [hblk, pblk]
    if tables:
        in_specs += [w((D, MSG)), w((D, MSG))]
        out_shape += [jax.ShapeDtypeStruct((N, TW), _f32),
                      jax.ShapeDtypeStruct((N, TW), _f32)]
        out_specs += [tblk, tblk]
    return pl.pallas_call(
        body, grid=(grid,), in_specs=in_specs,
        out_specs=out_specs, out_shape=out_shape)


_node_mid = _make_node_call(True)
_node_last = _make_node_call(False)


def _prologue_body(h_ref, pos_ref, ew, eb, wa, wb, h_out, tcol_out, trow_out):
    h0 = _dot(h_ref[...], ew[...]) + eb[...]
    h_out[...] = h0
    padz = jnp.zeros((h0.shape[0], TW - MSG - 3), _f32)
    pos = pos_ref[...]
    tcol_out[...] = jnp.concatenate([_dot(h0, wa[...]), pos, padz], axis=1)
    trow_out[...] = jnp.concatenate([_dot(h0, wb[...]), pos, padz], axis=1)


def _make_prologue():
    grid = N // NB
    hblk = pl.BlockSpec((NB, D), lambda i: (i, 0))
    pblk = pl.BlockSpec((NB, 3), lambda i: (i, 0))
    tblk = pl.BlockSpec((NB, TW), lambda i: (i, 0))

    def w(shape):
        return pl.BlockSpec(shape, lambda i: tuple(0 for _ in shape))

    return pl.pallas_call(
        _prologue_body, grid=(grid,),
        in_specs=[hblk, pblk, w((D, D)), w((1, D)), w((D, MSG)), w((D, MSG))],
        out_specs=[hblk, tblk, tblk],
        out_shape=[jax.ShapeDtypeStruct((N, D), _f32),
                   jax.ShapeDtypeStruct((N, TW), _f32),
                   jax.ShapeDtypeStruct((N, TW), _f32)])


_prologue = _make_prologue()


def kernel(h, pos, edge_index, embed_W, embed_b, msg_W1, msg_b1, msg_W2,
           msg_b2, coord_W1, coord_b1, coord_W2, node_W1, node_b1, node_W2,
           node_b2, ln_g, ln_b):
    row2 = jnp.concatenate([edge_index[0],
                            jnp.zeros((E2 - E,), jnp.int32)]).reshape(NCH2, CH)
    col2g = jnp.concatenate([edge_index[1],
                             jnp.zeros((E2 - E,), jnp.int32)]).reshape(NCH2, CH)
    col2s = jnp.concatenate([edge_index[1],
                             jnp.full((E2 - E,), N, jnp.int32)]).reshape(NCH2, CH)
    freqs = jnp.exp(-math.log(10000.0)
                    * jnp.arange(0, ED, 2, dtype=_f32) / ED)
    zeros_tbl = jnp.zeros((N, TW), _f32)

    W1a = msg_W1[:, :D]
    W1b = msg_W1[:, D:2 * D]
    W1c = jnp.concatenate([msg_W1[:, 2 * D:2 * D + ED:2],
                           msg_W1[:, 2 * D + 1:2 * D + ED:2]], axis=1)
    w1r = msg_W1[:, 2 * D + ED]

    c128 = jnp.arange(128)
    m32 = c128 % 32
    grp = c128 // 32
    in_msg = m32 < MSG
    in_cd = (m32 >= MSG) & (m32 < MSG + 3)
    oh4 = (jnp.arange(4)[:, None] == grp[None, :]).astype(_f32)
    mS = in_msg.astype(_f32)[None]
    mD = in_cd.astype(_f32)[None]
    Mrad = (oh4.T * in_cd[:, None].astype(_f32))
    E4 = oh4 * mD
    onesP = (m32 == MSG + 3).astype(_f32)[None]
    mclip = jnp.minimum(m32, MSG - 1)
    c64 = jnp.arange(64)
    r16 = c64 % 16
    g16 = c64 // 16
    oh16 = (jnp.arange(4)[:, None] == g16[None, :]).astype(_f32)
    F4 = oh16 * freqs[c64 % 8][None, :]
    cosOff = (math.pi / 2.0) * ((c64 % 16) >= 8).astype(_f32)[None]
    bd_mask = ((grp[:, None] == grp[None, :]) & in_msg[:, None]
               & in_msg[None, :]).astype(_f32)

    def bc24(v):
        return (jnp.where(in_msg, v[mclip], 0.0)).reshape(1, 128)

    def bd(Wsmall):
        return Wsmall[mclip][:, mclip] * bd_mask

    Wr4_l = [oh4 * bc24(w1r[l]) for l in range(L)]
    b1bc_l = [bc24(msg_b1[l]) for l in range(L)]
    b2bc_l = [bc24(msg_b2[l]) for l in range(L)]
    W2bd_l = [bd(msg_W2[l]) for l in range(L)]
    C1bd_l = [bd(coord_W1[l]) for l in range(L - 1)]
    cb1bc_l = [bc24(coord_b1[l]) for l in range(L - 1)]
    C2p_l = [(oh4.T * jnp.where(in_msg, coord_W2[l][mclip, 0], 0.0)[:, None])
             for l in range(L - 1)]
    wc_mask = (g16[:, None] == grp[None, :]).astype(_f32) * mS
    Wc64_l = [W1c[l][r16][:, mclip] * wc_mask for l in range(L)]
    nW1h = node_W1[:, :D]
    nW1m = node_W1[:, D:]
    nb1 = node_b1.reshape(L, 1, MSG)
    nb2 = node_b2.reshape(L, 1, D)
    g = ln_g.reshape(L, 1, D)
    bb = ln_b.reshape(L, 1, D)

    hcur, tcol, trow = _prologue(h, pos, embed_W, embed_b.reshape(1, D),
                                 W1a[0], W1b[0])
    poscur = pos
    ea = None
    for l in range(L):
        gcol, grow = _gather_k(tcol, trow, col2g, row2)
        gcolP = gcol.reshape(E2 // 4, 128)
        growP = grow.reshape(E2 // 4, 128)
        if l == 0:
            OP, ea = _edge_first(gcolP, growP, mS, mD, Mrad, Wr4_l[l],
                                 b1bc_l[l], W2bd_l[l], b2bc_l[l], C1bd_l[l],
                                 cb1bc_l[l], C2p_l[l], E4, onesP, Wc64_l[l],
                                 F4, cosOff)
        elif l < L - 1:
            OP = _edge_mid(gcolP, growP, ea, mS, mD, Mrad, Wr4_l[l],
                           b1bc_l[l], W2bd_l[l], b2bc_l[l], C1bd_l[l],
                           cb1bc_l[l], C2p_l[l], E4, onesP, Wc64_l[l])
        else:
            OP = _edge_last(gcolP, growP, ea, mS, mD, Mrad, Wr4_l[l],
                            b1bc_l[l], W2bd_l[l], b2bc_l[l], C1bd_l[0],
                            cb1bc_l[0], C2p_l[0], E4, onesP, Wc64_l[l])
        P = _scatter_k(OP.reshape(E2, TW), col2s, zeros_tbl)
        if l < L - 1:
            hcur, poscur, tcol, trow = _node_mid(
                hcur, poscur, P, nW1h[l], nW1m[l], nb1[l], node_W2[l],
                nb2[l], g[l], bb[l], W1a[l + 1], W1b[l + 1])
        else:
            hcur, poscur = _node_last(
                hcur, poscur, P, nW1h[l], nW1m[l], nb1[l], node_W2[l],
                nb2[l], g[l], bb[l])
    return hcur, poscur


# speedup vs baseline: 9.3297x; 1.0930x over previous
import functools
import math

import jax
import jax.numpy as jnp
from jax import lax
from jax.experimental import pallas as pl
from jax.experimental.pallas import tpu as pltpu
from jax.experimental.pallas import tpu_sc as plsc

N = 10000
E = 160000
D = 128
MSG = 24
ED = 16
L = 6
TW = 32
CH = 128
NC, NS = 2, 16
NW = NC * NS
E2 = 163840
NCH2 = E2 // CH
CPW = NCH2 // NW
SK = 4
EB = 4096
NB = 2000

_f32 = jnp.float32

_mesh = plsc.VectorSubcoreMesh(
    core_axis_name="c", subcore_axis_name="s", num_cores=NC, num_subcores=NS)


NSUP = CPW // SK


@functools.partial(
    pl.kernel,
    out_type=(jax.ShapeDtypeStruct((E2, TW), _f32),
              jax.ShapeDtypeStruct((E2, TW), _f32)),
    mesh=_mesh,
    scratch_types=[pltpu.VMEM((CPW, CH), jnp.int32),
                   pltpu.VMEM((CPW, CH), jnp.int32),
                   pltpu.VMEM((2, SK * CH, TW), _f32),
                   pltpu.VMEM((2, SK * CH, TW), _f32),
                   pltpu.SemaphoreType.DMA,
                   pltpu.SemaphoreType.DMA,
                   pltpu.SemaphoreType.DMA],
    compiler_params=pltpu.CompilerParams(use_tc_tiling_on_sc=False),
)
def _gather_k(tcol_hbm, trow_hbm, col2_hbm, row2_hbm, gcol_hbm, grow_hbm,
              idxc_v, idxr_v, bufc_v, bufr_v, gsem, wsemc, wsemr):
    cid = lax.axis_index("c")
    sid = lax.axis_index("s")
    wid = sid * NC + cid
    c0 = wid * CPW
    e0 = c0 * CH
    pltpu.sync_copy(col2_hbm.at[pl.ds(c0, CPW)], idxc_v)
    pltpu.sync_copy(row2_hbm.at[pl.ds(c0, CPW)], idxr_v)

    def body(s, carry):
        b = s % 2
        eo = e0 + s * SK * CH

        @pl.when(s >= 2)
        def _():
            pltpu.make_async_copy(
                bufc_v.at[b], gcol_hbm.at[pl.ds(eo, SK * CH)], wsemc).wait()
            pltpu.make_async_copy(
                bufr_v.at[b], grow_hbm.at[pl.ds(eo, SK * CH)], wsemr).wait()

        ds_ = []
        for k in range(SK):
            c = s * SK + k
            ds_.append(pltpu.async_copy(
                tcol_hbm.at[idxc_v.at[c]],
                bufc_v.at[b, pl.ds(k * CH, CH)], gsem))
            ds_.append(pltpu.async_copy(
                trow_hbm.at[idxr_v.at[c]],
                bufr_v.at[b, pl.ds(k * CH, CH)], gsem))
        for d in ds_:
            d.wait()
        pltpu.async_copy(bufc_v.at[b], gcol_hbm.at[pl.ds(eo, SK * CH)], wsemc)
        pltpu.async_copy(bufr_v.at[b], grow_hbm.at[pl.ds(eo, SK * CH)], wsemr)
        return carry

    lax.fori_loop(0, NSUP, body, 0)
    for s in (NSUP - 2, NSUP - 1):
        eo = e0 + s * SK * CH
        pltpu.make_async_copy(
            bufc_v.at[s % 2], gcol_hbm.at[pl.ds(eo, SK * CH)], wsemc).wait()
        pltpu.make_async_copy(
            bufr_v.at[s % 2], grow_hbm.at[pl.ds(eo, SK * CH)], wsemr).wait()


@functools.partial(
    pl.kernel,
    out_type=jax.ShapeDtypeStruct((NC, N, TW), _f32),
    mesh=_mesh,
    scratch_types=[pltpu.VMEM((CPW, CH), jnp.int32),
                   pltpu.VMEM((2, SK * CH, TW), _f32),
                   pltpu.VMEM_SHARED((N + 8, TW), _f32),
                   pltpu.SemaphoreType.DMA],
    compiler_params=pltpu.CompilerParams(use_tc_tiling_on_sc=False),
)
def _scatter_k(o_hbm, col2_hbm, zero_hbm, p_hbm, idx2_v, obuf_v, acc_sh, lsem):
    cid = lax.axis_index("c")
    sid = lax.axis_index("s")
    rows_per = N // NS
    r0 = sid * rows_per
    pltpu.sync_copy(zero_hbm.at[pl.ds(r0, rows_per)],
                    acc_sh.at[pl.ds(r0, rows_per)])
    c0 = cid * (NCH2 // NC) + sid * CPW
    e0 = c0 * CH
    pltpu.sync_copy(col2_hbm.at[pl.ds(c0, CPW)], idx2_v)
    pltpu.async_copy(o_hbm.at[pl.ds(e0, SK * CH)], obuf_v.at[0], lsem)
    plsc.subcore_barrier()

    def body(s, carry):
        b = s % 2
        pltpu.make_async_copy(
            o_hbm.at[pl.ds(e0, SK * CH)], obuf_v.at[b], lsem).wait()

        @pl.when(s + 1 < NSUP)
        def _():
            pltpu.async_copy(o_hbm.at[pl.ds(e0 + (s + 1) * SK * CH, SK * CH)],
                             obuf_v.at[(s + 1) % 2], lsem)
        for k in range(SK):
            c = s * SK + k
            pltpu.sync_copy(obuf_v.at[b, pl.ds(k * CH, CH)],
                            acc_sh.at[idx2_v.at[c]], add=True)
        return carry

    lax.fori_loop(0, NSUP, body, 0)
    plsc.subcore_barrier()
    pltpu.sync_copy(acc_sh.at[pl.ds(r0, rows_per)],
                    p_hbm.at[cid, pl.ds(r0, rows_per)])


def _silu(x):
    return x / (1.0 + jnp.exp(-x))


def _dot(a, b):
    return jnp.dot(a, b, preferred_element_type=_f32)


_W_R4, _W_B1, _W_B2, _W_CB1, _W_C2, _W_C64, _W_W2, _W_C1 = (
    0, 8, 16, 24, 32, 40, 104, 232)
WROWS = 360


def _edge_body(first, coords, refs):
    if first:
        gcol, grow, wp, o_ref, ea_out = refs
    else:
        gcol, grow, ea_in, wp, o_ref = refs
    Wr4 = wp[0, _W_R4:_W_R4 + 4, :]
    b1bc = wp[0, _W_B1:_W_B1 + 1, :]
    b2bc = wp[0, _W_B2:_W_B2 + 1, :]
    cb1bc = wp[0, _W_CB1:_W_CB1 + 1, :]
    C2pT = wp[0, _W_C2:_W_C2 + 4, :]
    Wc64 = wp[0, _W_C64:_W_C64 + 64, :]
    W2bd = wp[0, _W_W2:_W_W2 + 128, :]
    C1bd = wp[0, _W_C1:_W_C1 + 128, :]
    c128 = lax.broadcasted_iota(jnp.int32, (1, 128), 1)
    m32 = c128 % 32
    mD = ((m32 >= MSG) & (m32 < MSG + 3)).astype(_f32)
    onesP = (m32 == MSG + 3).astype(_f32)
    rr = lax.broadcasted_iota(jnp.int32, (128, 4), 0)
    gg = lax.broadcasted_iota(jnp.int32, (128, 4), 1)
    Mrad = (((rr // 32) == gg) & ((rr % 32) >= MSG)
            & ((rr % 32) < MSG + 3)).astype(_f32)
    g4 = lax.broadcasted_iota(jnp.int32, (4, 128), 0)
    c4 = lax.broadcasted_iota(jnp.int32, (4, 128), 1)
    E4 = (((c4 // 32) == g4) & ((c4 % 32) >= MSG)
          & ((c4 % 32) < MSG + 3)).astype(_f32)
    S = gcol[...] + grow[...]
    Dd = grow[...] - gcol[...]
    mS = (m32 < MSG).astype(_f32)
    cdP = Dd * mD
    R4 = _dot(cdP * cdP, Mrad)
    if first:
        gf = lax.broadcasted_iota(jnp.int32, (4, 64), 0)
        cf = lax.broadcasted_iota(jnp.int32, (4, 64), 1)
        freqs = jnp.exp((cf % 16 % 8).astype(_f32)
                        * (-math.log(10000.0) * 2.0 / ED))
        F4 = jnp.where((cf // 16) == gf, freqs, 0.0)
        c64 = lax.broadcasted_iota(jnp.int32, (1, 64), 1)
        cosOff = (math.pi / 2.0) * ((c64 % 16) >= 8).astype(_f32)
        ang = _dot(jnp.sqrt(R4), F4) + cosOff
        eaP = jnp.sin(ang)
        ea_out[...] = eaP
    else:
        eaP = ea_in[...]
    z1 = S * mS + _dot(R4, Wr4) + _dot(eaP, Wc64) + b1bc
    m1 = _silu(z1)
    m2 = _silu(_dot(m1, W2bd) + b2bc)
    if coords:
        t = _silu(_dot(m2, C1bd) + cb1bc)
        cw4 = lax.dot_general(t, C2pT, (((1,), (1,)), ((), ())),
                              preferred_element_type=_f32)
        o_ref[...] = m2 + cdP * _dot(cw4, E4) + onesP
    else:
        o_ref[...] = m2 + onesP


def _make_edge_call(first, coords, l):
    def body(*refs):
        _edge_body(first, coords, refs)

    grid = E2 // EB
    EBR = EB // 4
    eblk = pl.BlockSpec((EBR, 128), lambda i: (i, 0))
    eablk = pl.BlockSpec((EBR, 64), lambda i: (i, 0))
    wblk = pl.BlockSpec((1, WROWS, 128), lambda i: (l, 0, 0))

    in_specs = [eblk, eblk]
    if not first:
        in_specs.append(eablk)
    in_specs.append(wblk)
    out_shape = [jax.ShapeDtypeStruct((E2 // 4, 128), _f32)]
    out_specs = [eblk]
    if first:
        out_shape.append(jax.ShapeDtypeStruct((E2 // 4, 64), _f32))
        out_specs.append(eablk)
    return pl.pallas_call(
        body, grid=(grid,), in_specs=in_specs,
        out_specs=out_specs if len(out_specs) > 1 else out_specs[0],
        out_shape=out_shape if len(out_shape) > 1 else out_shape[0])


_edge_calls = [_make_edge_call(l == 0, l < L - 1, l) for l in range(L)]


def _node_body(tables, l, refs):
    if tables:
        (h_ref, pos_ref, p_ref, nw1, nb1, w2, nb2, g, b, wa, wb,
         h_out, pos_out, tcol_out, trow_out) = refs
    else:
        (h_ref, pos_ref, p_ref, nw1, nb1, w2, nb2, g, b,
         h_out, pos_out) = refs
    S = p_ref[0] + p_ref[1]
    msum = S[:, :MSG]
    coordagg = S[:, MSG:MSG + 3]
    cnt = S[:, MSG + 3:MSG + 4]
    agg = msum / jnp.maximum(cnt, 1.0)
    h = h_ref[...]
    z = _dot(h, nw1[0, :D, :]) + _dot(agg, nw1[0, D:, :]) + nb1[l:l + 1, :]
    h2 = h + _dot(_silu(z), w2[0]) + nb2[l:l + 1, :]
    mu = jnp.mean(h2, axis=-1, keepdims=True)
    hc = h2 - mu
    var = jnp.mean(hc * hc, axis=-1, keepdims=True)
    hn = hc / jnp.sqrt(var + 1e-5) * g[l:l + 1, :] + b[l:l + 1, :]
    h_out[...] = hn
    pos_new = pos_ref[...] + coordagg
    pos_out[...] = pos_new
    if tables:
        padz = jnp.zeros((hn.shape[0], TW - MSG - 3), _f32)
        tcol_out[...] = jnp.concatenate([_dot(hn, wa[0]), pos_new, padz],
                                        axis=1)
        trow_out[...] = jnp.concatenate([_dot(hn, wb[0]), pos_new, padz],
                                        axis=1)


def _make_node_call(l):
    tables = l < L - 1

    def body(*refs):
        _node_body(tables, l, refs)

    grid = N // NB
    hblk = pl.BlockSpec((NB, D), lambda i: (i, 0))
    pblk = pl.BlockSpec((NB, 3), lambda i: (i, 0))
    tblk = pl.BlockSpec((NB, TW), lambda i: (i, 0))
    partblk = pl.BlockSpec((NC, NB, TW), lambda i: (0, i, 0))

    in_specs = [hblk, pblk, partblk,
                pl.BlockSpec((1, D + MSG, MSG), lambda i: (l, 0, 0)),
                pl.BlockSpec((L, MSG), lambda i: (0, 0)),
                pl.BlockSpec((1, MSG, D), lambda i: (l, 0, 0)),
                pl.BlockSpec((L, D), lambda i: (0, 0)),
                pl.BlockSpec((L, D), lambda i: (0, 0)),
                pl.BlockSpec((L, D), lambda i: (0, 0))]
    out_shape = [jax.ShapeDtypeStruct((N, D), _f32),
                 jax.ShapeDtypeStruct((N, 3), _f32)]
    out_specs = [hblk, pblk]
    if tables:
        in_specs += [pl.BlockSpec((1, D, MSG), lambda i: (l + 1, 0, 0)),
                     pl.BlockSpec((1, D, MSG), lambda i: (l + 1, 1, 0))]
        out_shape += [jax.ShapeDtypeStruct((N, TW), _f32),
                      jax.ShapeDtypeStruct((N, TW), _f32)]
        out_specs += [tblk, tblk]
    return pl.pallas_call(
        body, grid=(grid,), in_specs=in_specs,
        out_specs=out_specs, out_shape=out_shape)


_node_calls = [_make_node_call(l) for l in range(L)]


def _prologue_body(h_ref, pos_ref, ew, eb, wa, wb, h_out, tcol_out, trow_out):
    h0 = _dot(h_ref[...], ew[...]) + eb[...]
    h_out[...] = h0
    padz = jnp.zeros((h0.shape[0], TW - MSG - 3), _f32)
    pos = pos_ref[...]
    tcol_out[...] = jnp.concatenate([_dot(h0, wa[0]), pos, padz], axis=1)
    trow_out[...] = jnp.concatenate([_dot(h0, wb[0]), pos, padz], axis=1)


def _make_prologue():
    grid = N // NB
    hblk = pl.BlockSpec((NB, D), lambda i: (i, 0))
    pblk = pl.BlockSpec((NB, 3), lambda i: (i, 0))
    tblk = pl.BlockSpec((NB, TW), lambda i: (i, 0))

    def w(shape):
        return pl.BlockSpec(shape, lambda i: tuple(0 for _ in shape))

    return pl.pallas_call(
        _prologue_body, grid=(grid,),
        in_specs=[hblk, pblk, w((D, D)), w((1, D)),
                  pl.BlockSpec((1, D, MSG), lambda i: (0, 0, 0)),
                  pl.BlockSpec((1, D, MSG), lambda i: (0, 1, 0))],
        out_specs=[hblk, tblk, tblk],
        out_shape=[jax.ShapeDtypeStruct((N, D), _f32),
                   jax.ShapeDtypeStruct((N, TW), _f32),
                   jax.ShapeDtypeStruct((N, TW), _f32)])


_prologue = _make_prologue()


def kernel(h, pos, edge_index, embed_W, embed_b, msg_W1, msg_b1, msg_W2,
           msg_b2, coord_W1, coord_b1, coord_W2, node_W1, node_b1, node_W2,
           node_b2, ln_g, ln_b):
    row2 = jnp.concatenate([edge_index[0],
                            jnp.zeros((E2 - E,), jnp.int32)]).reshape(NCH2, CH)
    col2g = jnp.concatenate([edge_index[1],
                             jnp.zeros((E2 - E,), jnp.int32)]).reshape(NCH2, CH)
    col2s = jnp.concatenate([edge_index[1],
                             jnp.full((E2 - E,), N, jnp.int32)]).reshape(NCH2, CH)
    zeros_tbl = jnp.zeros((N, TW), _f32)

    W1c = jnp.concatenate([msg_W1[:, 2 * D:2 * D + ED:2],
                           msg_W1[:, 2 * D + 1:2 * D + ED:2]], axis=1)
    w1r = msg_W1[:, 2 * D + ED]
    c128 = jnp.arange(128)
    m32 = c128 % 32
    grp = c128 // 32
    in_msg = m32 < MSG
    oh4 = (jnp.arange(4)[:, None] == grp[None, :]).astype(_f32)
    mclip = jnp.minimum(m32, MSG - 1)
    r16 = jnp.arange(64) % 16
    g16 = jnp.arange(64) // 16
    bd_mask = ((grp[:, None] == grp[None, :]) & in_msg[:, None]
               & in_msg[None, :]).astype(_f32)
    wc_mask = ((g16[:, None] == grp[None, :]).astype(_f32)
               * in_msg.astype(_f32)[None, :])
    cw1x = jnp.concatenate([coord_W1, coord_W1[:1]], 0)
    cb1x = jnp.concatenate([coord_b1, coord_b1[:1]], 0)
    cw2x = jnp.concatenate([coord_W2, coord_W2[:1]], 0)

    def bc24v(v):
        return jnp.where(in_msg[None, :], v[:, mclip], 0.0)[:, None, :]

    def zrows(n):
        return jnp.zeros((L, n, 128), _f32)

    wpack = jnp.concatenate([
        oh4[None] * bc24v(w1r), zrows(4),
        bc24v(msg_b1), zrows(7),
        bc24v(msg_b2), zrows(7),
        bc24v(cb1x), zrows(7),
        oh4[None] * bc24v(cw2x[:, :, 0]), zrows(4),
        W1c[:, r16][:, :, mclip] * wc_mask[None],
        msg_W2[:, mclip][:, :, mclip] * bd_mask[None],
        cw1x[:, mclip][:, :, mclip] * bd_mask[None],
    ], axis=1)

    hcur, tcol, trow = _prologue(h, pos, embed_W, embed_b.reshape(1, D),
                                 msg_W1, msg_W1)
    poscur = pos
    ea = None
    for l in range(L):
        gcol, grow = _gather_k(tcol, trow, col2g, row2)
        gcolP = gcol.reshape(E2 // 4, 128)
        growP = grow.reshape(E2 // 4, 128)
        if l == 0:
            OP, ea = _edge_calls[l](gcolP, growP, wpack)
        else:
            OP = _edge_calls[l](gcolP, growP, ea, wpack)
        P = _scatter_k(OP.reshape(E2, TW), col2s, zeros_tbl)
        if l < L - 1:
            hcur, poscur, tcol, trow = _node_calls[l](
                hcur, poscur, P, node_W1, node_b1, node_W2, node_b2,
                ln_g, ln_b, msg_W1, msg_W1)
        else:
            hcur, poscur = _node_calls[l](
                hcur, poscur, P, node_W1, node_b1, node_W2, node_b2,
                ln_g, ln_b)
    return hcur, poscur
